# baseline (speedup 1.0000x reference)
import numpy as np
import jax
import jax.numpy as jnp
from jax import lax
from jax.experimental import pallas as pl
from jax.experimental.pallas import tpu as pltpu

_BLK1 = 2000
_BLK2 = 2000
_N_BANK = 100000
_NB1 = _N_BANK // _BLK1
_NB2 = _N_BANK // _BLK2
_C = 64
_K = 9
_N_PIX = 784
_HW = 196
_BSZ = 4
_NCAND = 32
_NPB = _NCAND // _BSZ
_BIG = 3.0e38


def _resize_mat(inp=14, out=224):
    scale = inp / out
    x = (np.arange(out) + 0.5) * scale - 0.5
    j = np.arange(inp)
    w = np.maximum(0.0, 1.0 - np.abs(x[:, None] - j[None, :]))
    return w / w.sum(axis=1, keepdims=True)


def _blur_mat(n=224, sigma=4.0):
    r = int(round(4 * sigma))
    size = 2 * r + 1
    ax = np.arange(size) - r
    g = np.exp(-(ax * ax) / (2.0 * sigma * sigma))
    g = g / g.sum()
    G = np.zeros((n, n), np.float64)
    for i in range(n):
        lo = max(0, i - r)
        hi = min(n, i + r + 1)
        G[i, lo:hi] = g[(lo - i) + r:(hi - i) + r]
    return G


_A_MAT = np.ascontiguousarray((_blur_mat() @ _resize_mat()).astype(np.float32))
_AT_MAT = np.ascontiguousarray(_A_MAT.T)


def _aug(b):
    bb = jnp.sum(b * b, axis=1, keepdims=True)
    bbh = bb.astype(jnp.bfloat16)
    bbl = (bb - bbh.astype(jnp.float32)).astype(jnp.bfloat16)
    return jnp.concatenate([b.astype(jnp.bfloat16), bbh, bbl], axis=1)


def _mega_kern(at_ref, a2_ref, aa_ref, b_blk_ref, b_hbm, mine_ref, s_ref,
               bank_ref, rrows_ref, acc_ref, top_ref, nn_ref, cmin_ref,
               comb_ref, rsems):
    i = pl.program_id(0)

    b_aug = _aug(b_blk_ref[...])
    bank_ref[pl.ds(i * _BLK1, _BLK1), :] = b_aug
    e = lax.dot_general(b_aug, at_ref[...].astype(jnp.bfloat16),
                        (((1,), (0,)), ((), ())),
                        preferred_element_type=jnp.float32)
    m = jnp.min(e, axis=0, keepdims=True)

    @pl.when(i == 0)
    def _():
        acc_ref[...] = m

    @pl.when(i > 0)
    def _():
        acc_ref[...] = jnp.minimum(acc_ref[...], m)

    @pl.when(i == _NB1 - 1)
    def _():
        min_e = acc_ref[...]
        mine_ref[...] = min_e

        d2all = min_e + aa_ref[...]
        lane = lax.broadcasted_iota(jnp.int32, (1, _N_PIX), 1)
        batch_id = lane // _HW
        cand_idx = []
        for bb_ in range(_BSZ):
            dm = jnp.where(batch_id == bb_, d2all, -_BIG)
            for _ in range(_NPB):
                am = jnp.argmax(dm, axis=1)[0]
                cand_idx.append(am)
                dm = jnp.where(lane == am, -_BIG, dm)

        cand_rows = [a2_ref[pl.ds(ix, 1), :] for ix in cand_idx]
        candT = jnp.concatenate(cand_rows, axis=0)
        cand_bf = candT.astype(jnp.bfloat16)
        aa_list = [jnp.sum(jnp.where(lane == ix, aa_ref[...], 0.0), axis=1,
                           keepdims=True) for ix in cand_idx]
        aa32 = jnp.concatenate(aa_list, axis=1)

        top_ref[...] = jnp.full((16, _NCAND), _BIG, jnp.float32)
        nn_ref[...] = jnp.zeros((2, _NCAND), jnp.int32)
        cmin_ref[...] = jnp.full((1, _NCAND), _BIG, jnp.float32)
        srow = lax.broadcasted_iota(jnp.int32, (16 + _BLK2, _NCAND), 0)

        def p2_body(j, _):
            b2 = bank_ref[pl.ds(j * _BLK2, _BLK2), :]
            e2 = lax.dot_general(b2, cand_bf, (((1,), (1,)), ((), ())),
                                 preferred_element_type=jnp.float32)
            m2 = jnp.min(e2, axis=0, keepdims=True)
            am2 = jnp.argmin(e2, axis=0)[None, :] + j * _BLK2
            better = m2 < cmin_ref[...]
            nn_ref[0:1, :] = jnp.where(better, am2, nn_ref[0:1, :])
            cmin_ref[...] = jnp.minimum(cmin_ref[...], m2)

            @pl.when(jnp.any(m2 < top_ref[_K - 1:_K, :]))
            def _():
                comb_ref[0:16, :] = top_ref[...]
                comb_ref[16:, :] = e2
                for r in range(_K):
                    c = comb_ref[...]
                    top_ref[r:r + 1, :] = jnp.min(c, axis=0, keepdims=True)
                    amr = jnp.argmin(c, axis=0)[None, :]
                    comb_ref[...] = jnp.where(srow == amr, _BIG, c)
            return 0

        lax.fori_loop(0, _NB2, p2_body, 0)
        top = top_ref[...]
        nn = nn_ref[...]

        for cpos in range(_NCAND):
            row = nn[0, cpos]
            pltpu.make_async_copy(b_hbm.at[pl.ds(row, 1), :],
                                  rrows_ref.at[pl.ds(cpos, 1), :],
                                  rsems.at[cpos]).start()
        for cpos in range(_NCAND):
            row = nn[0, cpos]
            pltpu.make_async_copy(b_hbm.at[pl.ds(row, 1), :],
                                  rrows_ref.at[pl.ds(cpos, 1), :],
                                  rsems.at[cpos]).wait()
        d2x_cols = []
        for cpos in range(_NCAND):
            brow = rrows_ref[cpos:cpos + 1, :]
            arow = candT[cpos:cpos + 1, :_C]
            d2x_cols.append(jnp.sum(arow * brow + brow * brow, axis=1,
                                    keepdims=True))
        d2x = jnp.concatenate(d2x_cols, axis=1) + aa32

        conf0 = jnp.sqrt(jnp.maximum(d2x, 1e-12))
        conf_rest = jnp.sqrt(jnp.maximum(top[1:_K, :] + aa32, 1e-12))
        conf = jnp.concatenate([conf0, conf_rest], axis=0)
        ec = jnp.exp(conf)
        wgt = 1.0 - (jnp.max(ec, axis=0, keepdims=True) /
                     jnp.sum(ec, axis=0, keepdims=True))
        s32 = conf0 * wgt

        lane2 = lax.broadcasted_iota(jnp.int32, (1, _NCAND), 1)
        grp = lane2 // _NPB
        s_rows = []
        for bb_ in range(_BSZ):
            db = jnp.where(grp == bb_, d2x, -_BIG)
            vb = jnp.max(db)
            sb = jnp.max(jnp.where((grp == bb_) & (db == vb), s32, -_BIG))
            s_rows.append(jnp.full((1, 1), 1.0, jnp.float32) * sb)
        s_rows.append(jnp.zeros((4, 1), jnp.float32))
        s_ref[...] = jnp.concatenate(s_rows, axis=0)


def _mask_kern(d2_ref, a_ref, at_ref, o_ref):
    m14 = jnp.sqrt(jnp.maximum(d2_ref[0], 1e-12))
    t = jnp.dot(a_ref[...], m14, preferred_element_type=jnp.float32)
    o_ref[0] = jnp.dot(t, at_ref[...], preferred_element_type=jnp.float32)


def kernel(inputs, feature_vector):
    bsz, h, w, c = inputs.shape
    n_pix = bsz * h * w
    q = inputs.reshape(n_pix, c)
    aa = jnp.sum(q * q, axis=1)[None, :]
    a2 = jnp.concatenate([-2.0 * q, jnp.ones((n_pix, 2), jnp.float32)], axis=1)
    a_augT = a2.T

    min_e, s8 = pl.pallas_call(
        _mega_kern,
        grid=(_NB1,),
        in_specs=[
            pl.BlockSpec((c + 2, n_pix), lambda i: (0, 0)),
            pl.BlockSpec((n_pix, c + 2), lambda i: (0, 0)),
            pl.BlockSpec((1, n_pix), lambda i: (0, 0)),
            pl.BlockSpec((_BLK1, c), lambda i: (i, 0)),
            pl.BlockSpec(memory_space=pl.ANY),
        ],
        out_specs=[
            pl.BlockSpec((1, n_pix), lambda i: (0, 0)),
            pl.BlockSpec((8, 1), lambda i: (0, 0)),
        ],
        out_shape=[
            jax.ShapeDtypeStruct((1, n_pix), jnp.float32),
            jax.ShapeDtypeStruct((8, 1), jnp.float32),
        ],
        scratch_shapes=[
            pltpu.VMEM((_N_BANK, _C + 2), jnp.bfloat16),
            pltpu.VMEM((_NCAND, _C), jnp.float32),
            pltpu.VMEM((1, _N_PIX), jnp.float32),
            pltpu.VMEM((16, _NCAND), jnp.float32),
            pltpu.VMEM((2, _NCAND), jnp.int32),
            pltpu.VMEM((1, _NCAND), jnp.float32),
            pltpu.VMEM((16 + _BLK2, _NCAND), jnp.float32),
            pltpu.SemaphoreType.DMA((_NCAND,)),
        ],
    )(a_augT, a2, aa, feature_vector, feature_vector)

    s = s8[:bsz]
    d2min = (aa + min_e).reshape(bsz, h, w)

    mask = pl.pallas_call(
        _mask_kern,
        grid=(bsz,),
        in_specs=[
            pl.BlockSpec((1, h, w), lambda i: (i, 0, 0)),
            pl.BlockSpec((224, h), lambda i: (0, 0)),
            pl.BlockSpec((h, 224), lambda i: (0, 0)),
        ],
        out_specs=pl.BlockSpec((1, 224, 224), lambda i: (i, 0, 0)),
        out_shape=jax.ShapeDtypeStruct((bsz, 224, 224), jnp.float32),
    )(d2min, jnp.asarray(_A_MAT), jnp.asarray(_AT_MAT))

    return (s, mask.reshape(bsz, 224, 224, 1))


# speedup vs baseline: 2.0607x; 2.0607x over previous
import numpy as np
import jax
import jax.numpy as jnp
from jax import lax
from jax.experimental import pallas as pl

_BLK_A = 2000
_BLK_B = 10000
_N_BANK = 100000
_C = 64
_K = 9
_BIG = 3.0e38


def _resize_mat(inp=14, out=224):
    scale = inp / out
    x = (np.arange(out) + 0.5) * scale - 0.5
    j = np.arange(inp)
    w = np.maximum(0.0, 1.0 - np.abs(x[:, None] - j[None, :]))
    return w / w.sum(axis=1, keepdims=True)


def _blur_mat(n=224, sigma=4.0):
    r = int(round(4 * sigma))
    size = 2 * r + 1
    ax = np.arange(size) - r
    g = np.exp(-(ax * ax) / (2.0 * sigma * sigma))
    g = g / g.sum()
    G = np.zeros((n, n), np.float64)
    for i in range(n):
        lo = max(0, i - r)
        hi = min(n, i + r + 1)
        G[i, lo:hi] = g[(lo - i) + r:(hi - i) + r]
    return G


_A_MAT = np.ascontiguousarray((_blur_mat() @ _resize_mat()).astype(np.float32))
_AT_MAT = np.ascontiguousarray(_A_MAT.T)


def _min_kern(at_ref, b_ref, o_ref):
    i = pl.program_id(0)
    b = b_ref[...]
    bb = jnp.sum(b * b, axis=1, keepdims=True)
    b_aug = jnp.concatenate([b, bb], axis=1)
    e = lax.dot_general(b_aug, at_ref[...], (((1,), (0,)), ((), ())),
                        preferred_element_type=jnp.float32)
    m = jnp.min(e, axis=0, keepdims=True)

    @pl.when(i == 0)
    def _():
        o_ref[...] = m

    @pl.when(i > 0)
    def _():
        o_ref[...] = jnp.minimum(o_ref[...], m)


def _topk_kern(a4_ref, aa4_ref, b_ref, top_ref, s_ref):
    i = pl.program_id(0)
    nb = pl.num_programs(0)

    @pl.when(i == 0)
    def _():
        top_ref[...] = jnp.full((8, 16), _BIG, jnp.float32)

    b = b_ref[...]
    bb = jnp.sum(b * b, axis=1, keepdims=True)
    b_aug = jnp.concatenate([b, bb], axis=1)
    e = lax.dot_general(a4_ref[...], b_aug, (((1,), (1,)), ((), ())),
                        preferred_element_type=jnp.float32)
    m = jnp.min(e, axis=1, keepdims=True)

    @pl.when(jnp.any(m < top_ref[:, _K - 1:_K]))
    def _():
        comb = jnp.concatenate([top_ref[...], e], axis=1)
        iota = lax.broadcasted_iota(jnp.int32, comb.shape, 1)
        cols = []
        for _ in range(_K):
            v = jnp.min(comb, axis=1, keepdims=True)
            am = jnp.argmin(comb, axis=1)
            comb = jnp.where(iota == am[:, None], _BIG, comb)
            cols.append(v)
        cols.append(jnp.full((8, 16 - _K), _BIG, jnp.float32))
        top_ref[...] = jnp.concatenate(cols, axis=1)

    @pl.when(i == nb - 1)
    def _():
        conf = jnp.sqrt(jnp.maximum(top_ref[:, :_K] + aa4_ref[...], 1e-12))
        ec = jnp.exp(conf)
        w = 1.0 - jnp.max(ec, axis=1, keepdims=True) / jnp.sum(ec, axis=1, keepdims=True)
        s_ref[...] = conf[:, 0:1] * w


def _mask_kern(d2_ref, a_ref, at_ref, o_ref):
    m14 = jnp.sqrt(jnp.maximum(d2_ref[0], 1e-12))
    t = jnp.dot(a_ref[...], m14, preferred_element_type=jnp.float32)
    o_ref[0] = jnp.dot(t, at_ref[...], preferred_element_type=jnp.float32)


def kernel(inputs, feature_vector):
    bsz, h, w, c = inputs.shape
    n_pix = bsz * h * w
    q = inputs.reshape(n_pix, c)
    aa = jnp.sum(q * q, axis=1)
    a_aug = jnp.concatenate([-2.0 * q, jnp.ones((n_pix, 1), jnp.float32)], axis=1)
    a_augT = a_aug.T

    min_e = pl.pallas_call(
        _min_kern,
        grid=(_N_BANK // _BLK_A,),
        in_specs=[
            pl.BlockSpec((c + 1, n_pix), lambda i: (0, 0)),
            pl.BlockSpec((_BLK_A, c), lambda i: (i, 0)),
        ],
        out_specs=pl.BlockSpec((1, n_pix), lambda i: (0, 0)),
        out_shape=jax.ShapeDtypeStruct((1, n_pix), jnp.float32),
    )(a_augT, feature_vector)

    d2min = aa + min_e[0]
    idx = jnp.argmax(d2min.reshape(bsz, h * w), axis=1)
    sel = idx + jnp.arange(bsz) * (h * w)
    a4 = jnp.concatenate([a_aug[sel], jnp.zeros((8 - bsz, c + 1), jnp.float32)], axis=0)
    aa4 = jnp.concatenate([aa[sel], jnp.zeros((8 - bsz,), jnp.float32)])[:, None]

    _, s8 = pl.pallas_call(
        _topk_kern,
        grid=(_N_BANK // _BLK_B,),
        in_specs=[
            pl.BlockSpec((8, c + 1), lambda i: (0, 0)),
            pl.BlockSpec((8, 1), lambda i: (0, 0)),
            pl.BlockSpec((_BLK_B, c), lambda i: (i, 0)),
        ],
        out_specs=[
            pl.BlockSpec((8, 16), lambda i: (0, 0)),
            pl.BlockSpec((8, 1), lambda i: (0, 0)),
        ],
        out_shape=[
            jax.ShapeDtypeStruct((8, 16), jnp.float32),
            jax.ShapeDtypeStruct((8, 1), jnp.float32),
        ],
    )(a4, aa4, feature_vector)
    s = s8[:bsz]

    mask = pl.pallas_call(
        _mask_kern,
        grid=(bsz,),
        in_specs=[
            pl.BlockSpec((1, h, w), lambda i: (i, 0, 0)),
            pl.BlockSpec((224, h), lambda i: (0, 0)),
            pl.BlockSpec((h, 224), lambda i: (0, 0)),
        ],
        out_specs=pl.BlockSpec((1, 224, 224), lambda i: (i, 0, 0)),
        out_shape=jax.ShapeDtypeStruct((bsz, 224, 224), jnp.float32),
    )(d2min.reshape(bsz, h, w), jnp.asarray(_A_MAT), jnp.asarray(_AT_MAT))

    return (s, mask.reshape(bsz, 224, 224, 1))
